# baseline (speedup 1.0000x reference)
import jax
import jax.numpy as jnp
from jax import lax
from jax.experimental import pallas as pl
from jax.experimental.pallas import tpu as pltpu
from jax.experimental.pallas import tpu_sc as plsc

_NH = 1024
_DT = 0.1
_DELAY = 512
_K = 0.1

_ROWS = 2048
_COLS = 32768
_R = 64
_GRID = _ROWS // _R

_LANES = 16
_NW = 32
_CHUNK = _COLS // _NW


def _sc_body(buf_hbm, w_hbm, t_hbm, nx_hbm, tv, av, bv, xv, wv, nxv):
    c = lax.axis_index("c")
    s = lax.axis_index("s")
    wid = s * 2 + c
    base = wid * _CHUNK
    pltpu.sync_copy(t_hbm.at[0, pl.ds(0, _LANES)], tv)
    ts = tv[...][0]
    pltpu.sync_copy(buf_hbm.at[_NH + ts - _DELAY, pl.ds(base, _CHUNK)], av)
    pltpu.sync_copy(buf_hbm.at[_NH + ts + 1 - _DELAY, pl.ds(base, _CHUNK)], bv)
    pltpu.sync_copy(buf_hbm.at[_NH + ts, pl.ds(base, _CHUNK)], xv)
    pltpu.sync_copy(w_hbm.at[pl.ds(base, _CHUNK)], wv)
    for i in range(_CHUNK // _LANES):
        sl = pl.ds(i * _LANES, _LANES)
        x = xv[sl]
        a = av[sl]
        b = bv[sl]
        w = wv[sl]
        th1 = 1.0 - 2.0 / (jnp.exp(a * 2.0) + 1.0)
        d1 = -x + _K * th1
        xi = x + _DT * d1 + w
        th2 = 1.0 - 2.0 / (jnp.exp(b * 2.0) + 1.0)
        d2 = -xi + _K * th2
        nxv[sl] = x + _DT * 0.5 * (d1 + d2) + w
    pltpu.sync_copy(nxv, nx_hbm.at[pl.ds(base, _CHUNK)])


def _sc_nx(buf, dWt, t):
    mesh = plsc.VectorSubcoreMesh(core_axis_name="c", subcore_axis_name="s")
    return pl.kernel(
        _sc_body,
        out_type=jax.ShapeDtypeStruct((_COLS,), jnp.float32),
        mesh=mesh,
        scratch_types=[
            pltpu.VMEM((_LANES,), jnp.int32),
            pltpu.VMEM((_CHUNK,), jnp.float32),
            pltpu.VMEM((_CHUNK,), jnp.float32),
            pltpu.VMEM((_CHUNK,), jnp.float32),
            pltpu.VMEM((_CHUNK,), jnp.float32),
            pltpu.VMEM((_CHUNK,), jnp.float32),
        ],
    )(buf, dWt, t)


def _tc_body(ts_ref, buf_ref, nx_ref, outb_ref):
    ts = ts_ref[0]
    i = pl.program_id(0)
    outb_ref[...] = buf_ref[...]

    @pl.when(i == (_NH + ts + 1) // _R)
    def _patch():
        outb_ref[(_NH + ts + 1) % _R, :] = nx_ref[...]


def kernel(buf, dWt, t):
    nx = _sc_nx(buf, dWt, t)
    ts = t[0, 0:1].astype(jnp.int32)
    grid_spec = pltpu.PrefetchScalarGridSpec(
        num_scalar_prefetch=1,
        grid=(_GRID,),
        in_specs=[
            pl.BlockSpec((_R, _COLS), lambda i, ts: (i, 0)),
            pl.BlockSpec((_COLS,), lambda i, ts: (0,)),
        ],
        out_specs=[
            pl.BlockSpec((_R, _COLS), lambda i, ts: (i, 0)),
        ],
    )
    (buf2,) = pl.pallas_call(
        _tc_body,
        grid_spec=grid_spec,
        out_shape=[jax.ShapeDtypeStruct((_ROWS, _COLS), jnp.float32)],
    )(ts, buf, nx)
    return (buf2, nx)


# speedup vs baseline: 1.0357x; 1.0357x over previous
import jax
import jax.numpy as jnp
from jax import lax
from jax.experimental import pallas as pl
from jax.experimental.pallas import tpu as pltpu
from jax.experimental.pallas import tpu_sc as plsc

_NH = 1024
_DT = 0.1
_DELAY = 512
_K = 0.1

_ROWS = 2048
_COLS = 32768
_R = 64
_GRID = _ROWS // _R

_LANES = 16
_NW = 32
_CHUNK = _COLS // _NW


def _sc_body(buf_hbm, w_hbm, t_hbm, nx_hbm, tv, av, bv, xv, wv, nxv):
    c = lax.axis_index("c")
    s = lax.axis_index("s")
    wid = s * 2 + c
    base = wid * _CHUNK
    pltpu.sync_copy(t_hbm.at[0, pl.ds(0, _LANES)], tv)
    ts = tv[...][0]
    pltpu.sync_copy(buf_hbm.at[_NH + ts - _DELAY, pl.ds(base, _CHUNK)], av)
    pltpu.sync_copy(buf_hbm.at[_NH + ts + 1 - _DELAY, pl.ds(base, _CHUNK)], bv)
    pltpu.sync_copy(buf_hbm.at[_NH + ts, pl.ds(base, _CHUNK)], xv)
    pltpu.sync_copy(w_hbm.at[pl.ds(base, _CHUNK)], wv)
    for i in range(_CHUNK // _LANES):
        sl = pl.ds(i * _LANES, _LANES)
        x = xv[sl]
        a = av[sl]
        b = bv[sl]
        w = wv[sl]
        th1 = 1.0 - 2.0 / (jnp.exp(a * 2.0) + 1.0)
        d1 = -x + _K * th1
        xi = x + _DT * d1 + w
        th2 = 1.0 - 2.0 / (jnp.exp(b * 2.0) + 1.0)
        d2 = -xi + _K * th2
        nxv[sl] = x + _DT * 0.5 * (d1 + d2) + w
    pltpu.sync_copy(nxv, nx_hbm.at[pl.ds(base, _CHUNK)])


def _sc_nx(buf, dWt, t):
    mesh = plsc.VectorSubcoreMesh(core_axis_name="c", subcore_axis_name="s")
    return pl.kernel(
        _sc_body,
        out_type=jax.ShapeDtypeStruct((_COLS,), jnp.float32),
        mesh=mesh,
        scratch_types=[
            pltpu.VMEM((_LANES,), jnp.int32),
            pltpu.VMEM((_CHUNK,), jnp.float32),
            pltpu.VMEM((_CHUNK,), jnp.float32),
            pltpu.VMEM((_CHUNK,), jnp.float32),
            pltpu.VMEM((_CHUNK,), jnp.float32),
            pltpu.VMEM((_CHUNK,), jnp.float32),
        ],
    )(buf, dWt, t)


def _copy_body(buf_ref, outb_ref):
    outb_ref[...] = buf_ref[...]


def _patch_body(ts_ref, bufw_ref, nx_ref, outb_ref):
    ts = ts_ref[0]
    outb_ref[...] = bufw_ref[...]
    outb_ref[(_NH + ts + 1) % 8, :] = nx_ref[...]


def kernel(buf, dWt, t):
    nx = _sc_nx(buf, dWt, t)
    ts = t[0, 0:1].astype(jnp.int32)
    buf2a = pl.pallas_call(
        _copy_body,
        grid=(_GRID,),
        in_specs=[pl.BlockSpec((_R, _COLS), lambda i: (i, 0))],
        out_specs=pl.BlockSpec((_R, _COLS), lambda i: (i, 0)),
        out_shape=jax.ShapeDtypeStruct((_ROWS, _COLS), jnp.float32),
    )(buf)
    grid_spec = pltpu.PrefetchScalarGridSpec(
        num_scalar_prefetch=1,
        grid=(1,),
        in_specs=[
            pl.BlockSpec((8, _COLS), lambda i, ts: ((_NH + ts[0] + 1) // 8, 0)),
            pl.BlockSpec((_COLS,), lambda i, ts: (0,)),
        ],
        out_specs=[
            pl.BlockSpec((8, _COLS), lambda i, ts: ((_NH + ts[0] + 1) // 8, 0)),
        ],
    )
    (buf2,) = pl.pallas_call(
        _patch_body,
        grid_spec=grid_spec,
        out_shape=[jax.ShapeDtypeStruct((_ROWS, _COLS), jnp.float32)],
        input_output_aliases={1: 0},
    )(ts, buf2a, nx)
    return (buf2, nx)


# speedup vs baseline: 1.1459x; 1.1064x over previous
import jax
import jax.numpy as jnp
from jax.experimental import pallas as pl
from jax.experimental.pallas import tpu as pltpu

_NH = 1024
_DT = 0.1
_DELAY = 512
_K = 0.1

_ROWS = 2048
_COLS = 32768
_R = 64
_GRID = _ROWS // _R


def _body(ts_ref, buf_ref, w_ref, outb_ref, outnx_ref, ab_ref, nx_ref):
    ts = ts_ref[0]
    i = pl.program_id(0)
    outb_ref[...] = buf_ref[...]

    ra = _NH + ts - _DELAY
    rb = ra + 1
    rx = _NH + ts
    rp = rx + 1

    @pl.when(i == ra // _R)
    def _cap_a():
        ab_ref[0, :] = buf_ref[ra % _R, :]

    @pl.when(i == rb // _R)
    def _cap_b():
        ab_ref[1, :] = buf_ref[rb % _R, :]

    @pl.when(i == rx // _R)
    def _compute():
        x = buf_ref[rx % _R, :]
        w = w_ref[...]
        d1 = -x + _K * jnp.tanh(ab_ref[0, :])
        xi = x + _DT * d1 + w
        d2 = -xi + _K * jnp.tanh(ab_ref[1, :])
        nx = x + _DT * 0.5 * (d1 + d2) + w
        outnx_ref[...] = nx
        nx_ref[...] = nx

    @pl.when(i == rp // _R)
    def _patch():
        outb_ref[rp % _R, :] = nx_ref[...]


def kernel(buf, dWt, t):
    ts = t[0, 0:1].astype(jnp.int32)
    grid_spec = pltpu.PrefetchScalarGridSpec(
        num_scalar_prefetch=1,
        grid=(_GRID,),
        in_specs=[
            pl.BlockSpec((_R, _COLS), lambda i, ts: (i, 0)),
            pl.BlockSpec((_COLS,), lambda i, ts: (0,)),
        ],
        out_specs=[
            pl.BlockSpec((_R, _COLS), lambda i, ts: (i, 0)),
            pl.BlockSpec((_COLS,), lambda i, ts: (0,)),
        ],
        scratch_shapes=[
            pltpu.VMEM((2, _COLS), jnp.float32),
            pltpu.VMEM((_COLS,), jnp.float32),
        ],
    )
    buf2, nx = pl.pallas_call(
        _body,
        grid_spec=grid_spec,
        out_shape=[
            jax.ShapeDtypeStruct((_ROWS, _COLS), jnp.float32),
            jax.ShapeDtypeStruct((_COLS,), jnp.float32),
        ],
    )(ts, buf, dWt)
    return (buf2, nx)
